# baseline (speedup 1.0000x reference)
import functools

import jax
import jax.numpy as jnp
from jax import lax
from jax.experimental import pallas as pl
from jax.experimental.pallas import tpu as pltpu
from jax.experimental.pallas import tpu_sc as plsc

N = 10000
NP = 10240
E = 320000
D = 128
NC = 2
NS = 16
NW = NC * NS
C = 32
CPT = 79
EP = NW * CPT * 4 * C

_mesh = plsc.VectorSubcoreMesh(core_axis_name="c", subcore_axis_name="s")


@functools.partial(
    pl.kernel,
    out_type=[
        jax.ShapeDtypeStruct((NC, NP, D), jnp.float32),
        jax.ShapeDtypeStruct((NW, 1, NP), jnp.float32),
    ],
    mesh=_mesh,
    compiler_params=pltpu.CompilerParams(needs_layout_passes=False),
    scratch_types=[
        pltpu.VMEM((CPT, 1, 4 * C), jnp.int32),
        pltpu.VMEM((CPT, 1, 4 * C), jnp.int32),
        pltpu.VMEM((C, D), jnp.float32),
        pltpu.VMEM((C, D), jnp.float32),
        pltpu.VMEM((C, D), jnp.float32),
        pltpu.VMEM((C, D), jnp.float32),
        pltpu.VMEM((NP,), jnp.float32),
        pltpu.VMEM_SHARED((NP, D), jnp.float32),
        pltpu.SemaphoreType.DMA,
        pltpu.SemaphoreType.DMA,
        pltpu.SemaphoreType.DMA,
        pltpu.SemaphoreType.DMA,
    ],
)
def _sc_aggregate(x_hbm, src_hbm, dst_hbm, psum, degs, src_v, dst_v, rb0, rb1,
                  rb2, rb3, deg_v, acc, gs0, gs1, gs2, gs3):
    c = lax.axis_index("c")
    s = lax.axis_index("s")
    wid = c * NS + s

    pltpu.sync_copy(src_hbm.at[pl.ds(wid * CPT, CPT)], src_v)
    pltpu.sync_copy(dst_hbm.at[pl.ds(wid * CPT, CPT)], dst_v)

    zeros16 = jnp.zeros((16,), jnp.float32)

    def _zdeg(j, carry):
        for j2 in range(8):
            deg_v[pl.ds(j * 128 + j2 * 16, 16)] = zeros16
        return carry

    lax.fori_loop(0, NP // 128, _zdeg, 0)

    def _zrow(i, carry):
        for j in range(D // 16):
            rb0[i, pl.ds(j * 16, 16)] = zeros16
            rb1[i, pl.ds(j * 16, 16)] = zeros16
        return carry

    lax.fori_loop(0, C, _zrow, 0)
    zrows = NP // NS
    for r in range(zrows // (2 * C)):
        zbase = pl.multiple_of(s * zrows + r * 2 * C, 8)
        pltpu.sync_copy(rb0, acc.at[pl.ds(zbase, C)])
        pltpu.sync_copy(rb1, acc.at[pl.ds(zbase + C, C)])
    plsc.subcore_barrier()

    ones16 = jnp.ones((16,), jnp.float32)

    bufs = ((0, rb0, gs0), (C, rb1, gs1), (2 * C, rb2, gs2), (3 * C, rb3, gs3))
    for off, rb, gs in bufs:
        pltpu.async_copy(x_hbm.at[src_v.at[0, 0, pl.ds(off, C)]], rb, gs)

    def _pair(i, carry):
        for off, rb, gs in bufs:
            for k in range(C // 16):
                idx16 = dst_v[i, 0, pl.ds(off + k * 16, 16)]
                plsc.addupdate_scatter(deg_v, [idx16], ones16)
            pltpu.make_async_copy(
                x_hbm.at[src_v.at[i, 0, pl.ds(off, C)]], rb, gs).wait()
            pltpu.sync_copy(rb, acc.at[dst_v.at[i, 0, pl.ds(off, C)]],
                            add=True)

            @pl.when(i < CPT - 1)
            def _prefetch(off=off, rb=rb, gs=gs):
                pltpu.async_copy(
                    x_hbm.at[src_v.at[i + 1, 0, pl.ds(off, C)]], rb, gs)

        return carry

    lax.fori_loop(0, CPT, _pair, 0)
    pltpu.sync_copy(deg_v, degs.at[wid, 0])
    plsc.subcore_barrier()

    for r in range(zrows // (2 * C)):
        base = pl.multiple_of(s * zrows + r * 2 * C, 8)
        pltpu.sync_copy(acc.at[pl.ds(base, C)], rb0)
        pltpu.sync_copy(acc.at[pl.ds(base + C, C)], rb1)
        pltpu.sync_copy(rb0, psum.at[c, pl.ds(base, C)])
        pltpu.sync_copy(rb1, psum.at[c, pl.ds(base + C, C)])


BR = 1024


def _deg_body(dg_ref, o_ref):
    o_ref[...] = jnp.sum(dg_ref[...], axis=0)


def _deg_reduce(degs):
    return pl.pallas_call(
        _deg_body,
        out_shape=jax.ShapeDtypeStruct((NP // D, D), jnp.float32),
    )(degs)


def _xwr_body(x_ref, wr_ref, b_ref, o_ref):
    dn = (((1,), (1,)), ((), ()))
    o_ref[...] = lax.dot_general(x_ref[...], wr_ref[...], dn,
                                 preferred_element_type=jnp.float32) + b_ref[...]


def _tc_xwr(x, W_r, b_l):
    return pl.pallas_call(
        _xwr_body,
        grid=(NP // BR,),
        in_specs=[
            pl.BlockSpec((BR, D), lambda i: (i, 0)),
            pl.BlockSpec((D, D), lambda i: (0, 0)),
            pl.BlockSpec((1, D), lambda i: (0, 0)),
        ],
        out_specs=pl.BlockSpec((BR, D), lambda i: (i, 0)),
        out_shape=jax.ShapeDtypeStruct((N, D), jnp.float32),
    )(x, W_r, b_l.reshape(1, D))


def _tc_body(p_ref, dg_ref, y_ref, wl_ref, o_ref):
    dn = (((1,), (1,)), ((), ()))
    summed = p_ref[0] + p_ref[1]
    mean = summed / jnp.maximum(dg_ref[...], 1.0)
    acc = lax.dot_general(mean, wl_ref[...], dn,
                          preferred_element_type=jnp.float32)
    o_ref[...] = jnp.maximum(acc + y_ref[...], 0.0)


def _tc_finish(psum, deg, y, W_l):
    return pl.pallas_call(
        _tc_body,
        grid=(NP // BR,),
        in_specs=[
            pl.BlockSpec((NC, BR, D), lambda i: (0, i, 0)),
            pl.BlockSpec((BR, 1), lambda i: (i, 0)),
            pl.BlockSpec((BR, D), lambda i: (i, 0)),
            pl.BlockSpec((D, D), lambda i: (0, 0)),
        ],
        out_specs=pl.BlockSpec((BR, D), lambda i: (i, 0)),
        out_shape=jax.ShapeDtypeStruct((N, D), jnp.float32),
    )(psum, deg, y, W_l)


def kernel(x, edge_index, batch, W_l, b_l, W_r):
    npad = EP - E
    pad_src = (jnp.arange(npad, dtype=jnp.int32) * 37) % N
    pad_dst = N + (jnp.arange(npad, dtype=jnp.int32) % 16)
    src = jnp.concatenate([edge_index[0].astype(jnp.int32), pad_src])
    dst = jnp.concatenate([edge_index[1].astype(jnp.int32), pad_dst])
    src = src.reshape(NW * CPT, 1, 4 * C)
    dst = dst.reshape(NW * CPT, 1, 4 * C)
    psum, degs = _sc_aggregate(x, src, dst)
    y = _tc_xwr(x, W_r, b_l)
    deg = _deg_reduce(degs.reshape(NW, NP // D, D)).reshape(NP, 1)
    return _tc_finish(psum, deg, y, W_l)


# speedup vs baseline: 1.0026x; 1.0026x over previous
import functools

import jax
import jax.numpy as jnp
from jax import lax
from jax.experimental import pallas as pl
from jax.experimental.pallas import tpu as pltpu
from jax.experimental.pallas import tpu_sc as plsc

N = 10000
NP = 10240
E = 320000
D = 128
NC = 2
NS = 16
NW = NC * NS
C = 32
CPT = 79
EP = NW * CPT * 4 * C

_mesh = plsc.VectorSubcoreMesh(core_axis_name="c", subcore_axis_name="s")


@functools.partial(
    pl.kernel,
    out_type=[
        jax.ShapeDtypeStruct((NC, NP, D), jnp.float32),
        jax.ShapeDtypeStruct((NW, 1, NP), jnp.float32),
    ],
    mesh=_mesh,
    compiler_params=pltpu.CompilerParams(needs_layout_passes=False),
    scratch_types=[
        pltpu.VMEM((CPT, 1, 4 * C), jnp.int32),
        pltpu.VMEM((CPT, 1, 4 * C), jnp.int32),
        pltpu.VMEM((C, D), jnp.float32),
        pltpu.VMEM((C, D), jnp.float32),
        pltpu.VMEM((C, D), jnp.float32),
        pltpu.VMEM((C, D), jnp.float32),
        pltpu.VMEM((NP,), jnp.float32),
        pltpu.VMEM_SHARED((NP, D), jnp.float32),
        pltpu.SemaphoreType.DMA,
        pltpu.SemaphoreType.DMA,
        pltpu.SemaphoreType.DMA,
        pltpu.SemaphoreType.DMA,
    ],
)
def _sc_aggregate(x_hbm, src_hbm, dst_hbm, psum, degs, src_v, dst_v, rb0, rb1,
                  rb2, rb3, deg_v, acc, gs0, gs1, gs2, gs3):
    c = lax.axis_index("c")
    s = lax.axis_index("s")
    wid = c * NS + s

    pltpu.sync_copy(src_hbm.at[pl.ds(wid * CPT, CPT)], src_v)
    pltpu.sync_copy(dst_hbm.at[pl.ds(wid * CPT, CPT)], dst_v)

    zeros16 = jnp.zeros((16,), jnp.float32)

    def _zdeg(j, carry):
        for j2 in range(8):
            deg_v[pl.ds(j * 128 + j2 * 16, 16)] = zeros16
        return carry

    lax.fori_loop(0, NP // 128, _zdeg, 0)

    def _zrow(i, carry):
        for j in range(D // 16):
            rb0[i, pl.ds(j * 16, 16)] = zeros16
            rb1[i, pl.ds(j * 16, 16)] = zeros16
        return carry

    lax.fori_loop(0, C, _zrow, 0)
    zrows = NP // NS
    for r in range(zrows // (2 * C)):
        zbase = pl.multiple_of(s * zrows + r * 2 * C, 8)
        pltpu.sync_copy(rb0, acc.at[pl.ds(zbase, C)])
        pltpu.sync_copy(rb1, acc.at[pl.ds(zbase + C, C)])
    plsc.subcore_barrier()

    ones16 = jnp.ones((16,), jnp.float32)

    bufs = ((0, rb0, gs0), (C, rb1, gs1), (2 * C, rb2, gs2), (3 * C, rb3, gs3))
    for off, rb, gs in bufs:
        pltpu.async_copy(x_hbm.at[src_v.at[0, 0, pl.ds(off, C)]], rb, gs)

    def _pair(i, carry):
        for off, rb, gs in bufs:
            for k in range(C // 16):
                idx16 = dst_v[i, 0, pl.ds(off + k * 16, 16)]
                plsc.addupdate_scatter(deg_v, [idx16], ones16)
            pltpu.make_async_copy(
                x_hbm.at[src_v.at[i, 0, pl.ds(off, C)]], rb, gs).wait()
            pltpu.sync_copy(rb, acc.at[dst_v.at[i, 0, pl.ds(off, C)]],
                            add=True)

            @pl.when(i < CPT - 1)
            def _prefetch(off=off, rb=rb, gs=gs):
                pltpu.async_copy(
                    x_hbm.at[src_v.at[i + 1, 0, pl.ds(off, C)]], rb, gs)

        return carry

    lax.fori_loop(0, CPT, _pair, 0)
    pltpu.sync_copy(deg_v, degs.at[wid, 0])
    plsc.subcore_barrier()

    for r in range(zrows // (2 * C)):
        base = pl.multiple_of(s * zrows + r * 2 * C, 8)
        pltpu.sync_copy(acc.at[pl.ds(base, C)], rb0)
        pltpu.sync_copy(acc.at[pl.ds(base + C, C)], rb1)
        pltpu.sync_copy(rb0, psum.at[c, pl.ds(base, C)])
        pltpu.sync_copy(rb1, psum.at[c, pl.ds(base + C, C)])


BR = 1024


def _deg_body(dg_ref, o_ref):
    o_ref[...] = jnp.sum(dg_ref[...], axis=0)


def _deg_reduce(degs):
    return pl.pallas_call(
        _deg_body,
        out_shape=jax.ShapeDtypeStruct((NP // D, D), jnp.float32),
    )(degs)


def _tc_body(p_ref, dg_ref, x_ref, wl_ref, wr_ref, b_ref, o_ref):
    dn = (((1,), (1,)), ((), ()))
    summed = p_ref[0] + p_ref[1]
    mean = summed / jnp.maximum(dg_ref[...], 1.0)
    acc = lax.dot_general(mean, wl_ref[...], dn,
                          preferred_element_type=jnp.float32)
    acc = acc + lax.dot_general(x_ref[...], wr_ref[...], dn,
                                preferred_element_type=jnp.float32)
    o_ref[...] = jnp.maximum(acc + b_ref[...], 0.0)


def _tc_finish(psum, deg, x, W_l, b_l, W_r):
    return pl.pallas_call(
        _tc_body,
        grid=(NP // BR,),
        in_specs=[
            pl.BlockSpec((NC, BR, D), lambda i: (0, i, 0)),
            pl.BlockSpec((BR, 1), lambda i: (i, 0)),
            pl.BlockSpec((BR, D), lambda i: (i, 0)),
            pl.BlockSpec((D, D), lambda i: (0, 0)),
            pl.BlockSpec((D, D), lambda i: (0, 0)),
            pl.BlockSpec((1, D), lambda i: (0, 0)),
        ],
        out_specs=pl.BlockSpec((BR, D), lambda i: (i, 0)),
        out_shape=jax.ShapeDtypeStruct((N, D), jnp.float32),
    )(psum, deg, x, W_l, W_r, b_l.reshape(1, D))


def kernel(x, edge_index, batch, W_l, b_l, W_r):
    npad = EP - E
    pad_src = (jnp.arange(npad, dtype=jnp.int32) * 37) % N
    pad_dst = N + (jnp.arange(npad, dtype=jnp.int32) % 16)
    src = jnp.concatenate([edge_index[0].astype(jnp.int32), pad_src])
    dst = jnp.concatenate([edge_index[1].astype(jnp.int32), pad_dst])
    src = src.reshape(NW * CPT, 1, 4 * C)
    dst = dst.reshape(NW * CPT, 1, 4 * C)
    psum, degs = _sc_aggregate(x, src, dst)
    deg = _deg_reduce(degs.reshape(NW, NP // D, D)).reshape(NP, 1)
    return _tc_finish(psum, deg, x, W_l, b_l, W_r)
